# baseline (speedup 1.0000x reference)
import functools

import jax
import jax.numpy as jnp
from jax import lax
from jax.experimental import pallas as pl
from jax.experimental.pallas import tpu as pltpu
from jax.experimental.pallas import tpu_sc as plsc

_VOCAB = 1000
_D = 1000
_B = 4096 * 20
_NC = 2
_NS = 16
_NW = _NC * _NS
_BPW = _B // _NW
_G = 16
_NGRP = _BPW // _G

_mesh = plsc.VectorSubcoreMesh(core_axis_name="c", subcore_axis_name="s")


@functools.partial(
    pl.kernel,
    mesh=_mesh,
    out_type=jax.ShapeDtypeStruct((_B, _D), jnp.float32),
    scratch_types=[
        pltpu.VMEM((_BPW,), jnp.int32),
        pltpu.SemaphoreType.DMA((_G,)),
    ],
)
def _gather_kernel(x_hbm, table_hbm, out_hbm, idx_v, dsem):
    wid = lax.axis_index("s") * _NC + lax.axis_index("c")
    base = pl.multiple_of(wid * _BPW, 8)
    pltpu.sync_copy(x_hbm.at[pl.ds(base, _BPW)], idx_v)

    def row_copy(i, slot, r):
        pltpu.async_copy(
            table_hbm.at[pl.ds(r, 1)],
            out_hbm.at[pl.ds(base + i, 1)],
            dsem.at[slot],
        )

    def row_drain(slot):
        pltpu.make_async_copy(
            table_hbm.at[pl.ds(0, 1)],
            out_hbm.at[pl.ds(base, 1)],
            dsem.at[slot],
        ).wait()

    v0 = idx_v[pl.ds(0, _G)]
    for k in range(_G):
        row_copy(k, k, v0[k])

    def body(g, _):
        v = idx_v[pl.ds(g * _G, _G)]
        for k in range(_G):
            row_drain(k)
            row_copy(g * _G + k, k, v[k])
        return 0

    lax.fori_loop(1, _NGRP, body, 0)

    for k in range(_G):
        row_drain(k)


def kernel(x, table):
    x_flat = x.reshape(-1).astype(jnp.int32)
    out = _gather_kernel(x_flat, table)
    return out.reshape(x.shape[0], x.shape[1], _D)


# speedup vs baseline: 9.1833x; 9.1833x over previous
import functools

import jax
import jax.numpy as jnp
from jax import lax
from jax.experimental import pallas as pl
from jax.experimental.pallas import tpu as pltpu
from jax.experimental.pallas import tpu_sc as plsc

_VOCAB = 1000
_D = 1000
_DP = 1024
_B = 4096 * 20
_NC = 2
_NS = 16
_NW = _NC * _NS
_BPW = _B // _NW
_C = 16
_NCHUNK = _BPW // _C
_NBUF = 2

_mesh = plsc.VectorSubcoreMesh(core_axis_name="c", subcore_axis_name="s")


@functools.partial(
    pl.kernel,
    mesh=_mesh,
    out_type=jax.ShapeDtypeStruct((_B, _D), jnp.float32),
    scratch_types=[
        pltpu.VMEM((_BPW,), jnp.int32),
        pltpu.VMEM((_NBUF, _C, 8, 128), jnp.float32),
        pltpu.VMEM((_NBUF, _C, _D), jnp.float32),
        pltpu.SemaphoreType.DMA((_NBUF,)),
        pltpu.SemaphoreType.DMA((_NBUF,)),
    ],
)
def _gather_kernel(x_hbm, table_hbm, out_hbm, idx_v, rows_v, rb_v,
                   gsem, ssem):
    wid = lax.axis_index("s") * _NC + lax.axis_index("c")
    base = pl.multiple_of(wid * _BPW, 8)
    pltpu.sync_copy(x_hbm.at[pl.ds(base, _BPW)], idx_v)

    def gather_start(g, slot):
        off = pl.multiple_of(g * _C, 8)
        pltpu.async_copy(
            table_hbm.at[idx_v.at[pl.ds(off, _C)]],
            rows_v.at[slot],
            gsem.at[slot],
        )

    def gather_wait(slot):
        pltpu.make_async_copy(
            table_hbm.at[idx_v.at[pl.ds(0, _C)]],
            rows_v.at[slot],
            gsem.at[slot],
        ).wait()

    def out_start(g, slot):
        pltpu.async_copy(
            rb_v.at[slot],
            out_hbm.at[pl.ds(base + g * _C, _C)],
            ssem.at[slot],
        )

    def out_wait(slot):
        pltpu.make_async_copy(
            rb_v.at[slot],
            out_hbm.at[pl.ds(base, _C)],
            ssem.at[slot],
        ).wait()

    lane = lax.iota(jnp.int32, 16)

    def rearrange(slot):
        def per_row(p, _):
            for j in range(7):
                for v in range(0, 128, 16):
                    seg = rows_v[slot, p, j, pl.ds(v, 16)]
                    rb_v[slot, p, pl.ds(128 * j + v, 16)] = seg
            for v in range(0, 96, 16):
                seg = rows_v[slot, p, 7, pl.ds(v, 16)]
                rb_v[slot, p, pl.ds(896 + v, 16)] = seg
            seg = rows_v[slot, p, 7, pl.ds(96, 16)]
            rb_v[slot, p, pl.ds(992, 8)] = lax.slice(seg, (0,), (8,))
            return 0

        lax.fori_loop(0, _C, per_row, 0)

    for b in range(_NBUF):
        gather_start(b, b)

    def body(g, _):
        slot = lax.rem(g, _NBUF)
        gather_wait(slot)

        @pl.when(g >= _NBUF)
        def _():
            out_wait(slot)

        rearrange(slot)
        out_start(g, slot)

        @pl.when(g + _NBUF < _NCHUNK)
        def _():
            gather_start(g + _NBUF, slot)

        return 0

    lax.fori_loop(0, _NCHUNK, body, 0)

    for b in range(_NBUF):
        out_wait(b)


def kernel(x, table):
    x_flat = x.reshape(-1).astype(jnp.int32)
    table3 = jnp.pad(table, ((0, 0), (0, _DP - _D))).reshape(_VOCAB, 8, 128)
    out = _gather_kernel(x_flat, table3)
    return out.reshape(x.shape[0], x.shape[1], _D)
